# baseline (speedup 1.0000x reference)
import functools

import jax
import jax.numpy as jnp
from jax import lax
from jax.experimental import pallas as pl
from jax.experimental.pallas import tpu as pltpu
from jax.experimental.pallas import tpu_sc as plsc

_N_VOCAB = 1000000
_HALF = _N_VOCAB // 2
_N_EMBED = 64
_BATCH = 16384

_NUM_CORES = 2
_NUM_SUBCORES = 16
_NUM_WORKERS = _NUM_CORES * _NUM_SUBCORES
_B_PER_W = _BATCH // _NUM_WORKERS

_mesh = plsc.VectorSubcoreMesh(core_axis_name="c", subcore_axis_name="s")


def _make_half_gather(base):
    @functools.partial(
        pl.kernel,
        mesh=_mesh,
        out_type=jax.ShapeDtypeStruct((_BATCH, _N_EMBED), jnp.float32),
        scratch_types=[
            pltpu.VMEM((_B_PER_W,), jnp.int32),
            pltpu.VMEM((_B_PER_W, _N_EMBED), jnp.float32),
            pltpu.SemaphoreType.DMA,
        ],
    )
    def _half_gather(idx_hbm, table_hbm, out_hbm, idx_v, rows_v, sem):
        wid = lax.axis_index("s") * _NUM_CORES + lax.axis_index("c")
        wbase = wid * _B_PER_W
        pltpu.sync_copy(idx_hbm.at[pl.ds(wbase, _B_PER_W)], idx_v)

        zeros = jnp.zeros((16,), jnp.int32)

        def fire(g, carry):
            v = idx_v[pl.ds(g * 16, 16)] - base
            ok = (v >= 0) & (v < _HALF)
            cv = jnp.where(ok, v, zeros)
            for j in range(16):
                p = cv[j]
                pltpu.async_copy(
                    table_hbm.at[pl.ds(p, 1)],
                    rows_v.at[pl.ds(g * 16 + j, 1)],
                    sem,
                )
            return carry

        lax.fori_loop(0, _B_PER_W // 16, fire, 0)

        def drain(i, carry):
            pltpu.make_async_copy(
                table_hbm.at[pl.ds(0, 1)],
                rows_v.at[pl.ds(0, 1)],
                sem,
            ).wait()
            return carry

        lax.fori_loop(0, _B_PER_W, drain, 0)

        pltpu.sync_copy(rows_v, out_hbm.at[pl.ds(wbase, _B_PER_W)])

    return _half_gather


_gather_lo = _make_half_gather(0)
_gather_hi = _make_half_gather(_HALF)


def kernel(input_words, in_embed):
    idx = input_words.astype(jnp.int32)
    out_lo = _gather_lo(idx, in_embed[:_HALF])
    out_hi = _gather_hi(idx, in_embed[_HALF:])
    return jnp.where((idx < _HALF)[:, None], out_lo, out_hi)


# speedup vs baseline: 2.2756x; 2.2756x over previous
import functools

import jax
import jax.numpy as jnp
from jax import lax
from jax.experimental import pallas as pl
from jax.experimental.pallas import tpu as pltpu

_N_VOCAB = 1000000
_N_EMBED = 64
_BATCH = 16384
_UNROLL = 8


def _tc_gather(idx_ref, table_ref, out_ref, rows_v, sem):
    def fire(g, carry):
        for j in range(_UNROLL):
            k = g * _UNROLL + j
            p = idx_ref[k]
            pltpu.make_async_copy(
                table_ref.at[pl.ds(p, 1)],
                rows_v.at[pl.ds(k, 1)],
                sem,
            ).start()
        return carry

    lax.fori_loop(0, _BATCH // _UNROLL, fire, 0)

    def drain(i, carry):
        pltpu.make_async_copy(
            table_ref.at[pl.ds(0, 1)],
            rows_v.at[pl.ds(0, 1)],
            sem,
        ).wait()
        return carry

    lax.fori_loop(0, _BATCH, drain, 0)

    pltpu.make_async_copy(rows_v, out_ref, sem).start()
    pltpu.make_async_copy(rows_v, out_ref, sem).wait()


@jax.jit
def _gather(idx, table):
    return pl.pallas_call(
        _tc_gather,
        grid_spec=pltpu.PrefetchScalarGridSpec(
            num_scalar_prefetch=1,
            grid=(1,),
            in_specs=[pl.BlockSpec(memory_space=pl.ANY)],
            out_specs=pl.BlockSpec(memory_space=pl.ANY),
            scratch_shapes=[
                pltpu.VMEM((_BATCH, _N_EMBED), jnp.float32),
                pltpu.SemaphoreType.DMA,
            ],
        ),
        out_shape=jax.ShapeDtypeStruct((_BATCH, _N_EMBED), jnp.float32),
    )(idx, table)


def kernel(input_words, in_embed):
    idx = input_words.astype(jnp.int32)
    return _gather(idx, in_embed)


# speedup vs baseline: 3.1874x; 1.4007x over previous
import functools

import jax
import jax.numpy as jnp
from jax import lax
from jax.experimental import pallas as pl
from jax.experimental.pallas import tpu as pltpu
from jax.experimental.pallas import tpu_sc as plsc

_N_VOCAB = 1000000
_N_EMBED = 64
_BATCH = 16384

_NUM_CORES = 2
_NUM_SUBCORES = 16
_NUM_WORKERS = _NUM_CORES * _NUM_SUBCORES
_B_PER_W = _BATCH // _NUM_WORKERS

_mesh = plsc.VectorSubcoreMesh(core_axis_name="c", subcore_axis_name="s")


@functools.partial(
    pl.kernel,
    mesh=_mesh,
    out_type=jax.ShapeDtypeStruct((_BATCH, _N_EMBED), jnp.float32),
    scratch_types=[
        pltpu.VMEM((_B_PER_W,), jnp.int32),
        pltpu.VMEM((_B_PER_W, _N_EMBED), jnp.float32),
        pltpu.SemaphoreType.DMA,
    ],
)
def _sc_gather(idx_hbm, table_hbm, out_hbm, idx_v, rows_v, sem):
    wid = lax.axis_index("s") * _NUM_CORES + lax.axis_index("c")
    base = wid * _B_PER_W
    pltpu.sync_copy(idx_hbm.at[pl.ds(base, _B_PER_W)], idx_v)

    def fire(g, carry):
        v = idx_v[pl.ds(g * 16, 16)]
        for j in range(16):
            p = v[j]
            pltpu.async_copy(
                table_hbm.at[pl.ds(p, 1)],
                rows_v.at[pl.ds(g * 16 + j, 1)],
                sem,
            )
        return carry

    lax.fori_loop(0, _B_PER_W // 16, fire, 0)

    def drain(i, carry):
        pltpu.make_async_copy(
            table_hbm.at[pl.ds(0, 1)],
            rows_v.at[pl.ds(0, 1)],
            sem,
        ).wait()
        return carry

    lax.fori_loop(0, _B_PER_W, drain, 0)

    pltpu.sync_copy(rows_v, out_hbm.at[pl.ds(base, _B_PER_W)])


def kernel(input_words, in_embed):
    idx = input_words.astype(jnp.int32)
    return _sc_gather(idx, in_embed)
